# Initial kernel scaffold; baseline (speedup 1.0000x reference)
import jax
import jax.numpy as jnp
from jax.experimental import pallas as pl


def kernel(x, bias, Wq, bq, proto_key, Wv, bv, emb, alpha, beta):
    raise NotImplementedError("write your pallas kernel here")


# speedup vs baseline: 7.7155x; 7.7155x over previous
import functools
import jax
import jax.numpy as jnp
from jax import lax
from jax.experimental import pallas as pl
from jax.experimental.pallas import tpu as pltpu

N = 16384
DIM = 768
H = 12
HD = 64
P = 64
HP = H * P
K2 = 2
K1 = 8
SEG = 64
NSEG = N // SEG
TILE = 256
NTILE = N // TILE
SEG_PER_TILE = TILE // SEG

_NEG = -1e30


def _s1_body(x_ref, wq_ref, bq_ref, pkt_ref, bias_ref,
             attn_t_ref, segmax_t_ref, i1_ref, i2_ref, a1_ref, a2_ref):
    x = x_ref[...]
    q = jnp.dot(x.astype(jnp.bfloat16), wq_ref[...],
                preferred_element_type=jnp.float32) + bq_ref[...]
    qb = q.astype(jnp.bfloat16)
    ahs = []
    for h in range(H):
        ah = jnp.dot(qb[:, h * HD:(h + 1) * HD], pkt_ref[:, h * P:(h + 1) * P],
                     preferred_element_type=jnp.float32) * (HD ** -0.5)
        ahs.append(ah)
    a = jnp.concatenate(ahs, axis=1)
    attn_t_ref[...] = a.T
    segmax_t_ref[...] = a.reshape(SEG_PER_TILE, SEG, HP).max(axis=1).T[None]

    iota = lax.broadcasted_iota(jnp.int32, (TILE, P), 1)
    i1s, i2s, a1s, a2s = [], [], [], []
    for h in range(H):
        ah = ahs[h]
        sch = jax.nn.sigmoid(ah) + bias_ref[h:h + 1, :]
        m1 = sch.max(axis=1, keepdims=True)
        i1 = jnp.where(sch == m1, iota, P).min(axis=1, keepdims=True)
        a1 = jnp.where(iota == i1, ah, 0.0).sum(axis=1, keepdims=True)
        sc2 = jnp.where(iota == i1, _NEG, sch)
        m2 = sc2.max(axis=1, keepdims=True)
        i2 = jnp.where(sc2 == m2, iota, P).min(axis=1, keepdims=True)
        a2 = jnp.where(iota == i2, ah, 0.0).sum(axis=1, keepdims=True)
        i1s.append(i1); i2s.append(i2); a1s.append(a1); a2s.append(a2)
    i1_ref[...] = jnp.concatenate(i1s, axis=1)
    i2_ref[...] = jnp.concatenate(i2s, axis=1)
    a1_ref[...] = jnp.concatenate(a1s, axis=1)
    a2_ref[...] = jnp.concatenate(a2s, axis=1)


def _s1(x, wq_b, bq2, pkt_b, bias):
    return pl.pallas_call(
        _s1_body,
        grid=(NTILE,),
        in_specs=[
            pl.BlockSpec((TILE, DIM), lambda i: (i, 0)),
            pl.BlockSpec((DIM, DIM), lambda i: (0, 0)),
            pl.BlockSpec((1, DIM), lambda i: (0, 0)),
            pl.BlockSpec((HD, HP), lambda i: (0, 0)),
            pl.BlockSpec((H, P), lambda i: (0, 0)),
        ],
        out_specs=[
            pl.BlockSpec((HP, TILE), lambda i: (0, i)),
            pl.BlockSpec((1, HP, SEG_PER_TILE), lambda i: (i, 0, 0)),
            pl.BlockSpec((TILE, H), lambda i: (i, 0)),
            pl.BlockSpec((TILE, H), lambda i: (i, 0)),
            pl.BlockSpec((TILE, H), lambda i: (i, 0)),
            pl.BlockSpec((TILE, H), lambda i: (i, 0)),
        ],
        out_shape=[
            jax.ShapeDtypeStruct((HP, N), jnp.float32),
            jax.ShapeDtypeStruct((NTILE, HP, SEG_PER_TILE), jnp.float32),
            jax.ShapeDtypeStruct((N, H), jnp.int32),
            jax.ShapeDtypeStruct((N, H), jnp.int32),
            jax.ShapeDtypeStruct((N, H), jnp.float32),
            jax.ShapeDtypeStruct((N, H), jnp.float32),
        ],
    )(x, wq_b, bq2, pkt_b, bias)


def _s2_jax(attn_t, segmax_t, emb_c, x):
    t8val, t8idx = lax.top_k(attn_t, K1)
    sw = jax.nn.sigmoid(t8val + emb_c[:, None])
    xrows = x[t8idx]
    u = jnp.einsum('ck,ckf->cf', sw, xrows)
    ssum = sw.sum(-1)
    return u, ssum


def _s3_body(u_ref, wv_ref, bvtile_ref, ssum_ref, v1_ref):
    full = jnp.dot(u_ref[...], wv_ref[...], preferred_element_type=jnp.float32)
    blk = jnp.concatenate(
        [full[h * P:(h + 1) * P, h * HD:(h + 1) * HD] for h in range(H)], axis=0)
    v1_ref[...] = blk + ssum_ref[...] * bvtile_ref[...]


def _s3(u, wv, bvtile, ssum_col):
    return pl.pallas_call(
        _s3_body,
        in_specs=[
            pl.BlockSpec((HP, DIM), lambda: (0, 0)),
            pl.BlockSpec((DIM, DIM), lambda: (0, 0)),
            pl.BlockSpec((HP, HD), lambda: (0, 0)),
            pl.BlockSpec((HP, 1), lambda: (0, 0)),
        ],
        out_specs=pl.BlockSpec((HP, HD), lambda: (0, 0)),
        out_shape=jax.ShapeDtypeStruct((HP, HD), jnp.float32),
    )(u, wv, bvtile, ssum_col)


def _s4_body(x_ref, wv_ref, bv2_ref, v1_ref, i1_ref, i2_ref, a1_ref, a2_ref,
             sab_ref, out_ref):
    x = x_ref[...]
    xv = jnp.dot(x.astype(jnp.bfloat16), wv_ref[...],
                 preferred_element_type=jnp.float32) + bv2_ref[...]
    iota = lax.broadcasted_iota(jnp.int32, (TILE, P), 1)
    for h in range(H):
        i1 = i1_ref[:, h:h + 1]
        i2 = i2_ref[:, h:h + 1]
        a1 = a1_ref[:, h:h + 1]
        a2 = a2_ref[:, h:h + 1]
        mx = jnp.maximum(a1, a2)
        e1 = jnp.exp(a1 - mx)
        e2 = jnp.exp(a2 - mx)
        inv = 1.0 / (e1 + e2)
        w1 = e1 * inv
        w2 = e2 * inv
        whot = jnp.where(iota == i1, w1, 0.0) + jnp.where(iota == i2, w2, 0.0)
        vm = jnp.dot(whot.astype(jnp.bfloat16),
                     v1_ref[h * P:(h + 1) * P, :].astype(jnp.bfloat16),
                     preferred_element_type=jnp.float32)
        sa = sab_ref[0:1, h:h + 1]
        sb = sab_ref[1:2, h:h + 1]
        out_ref[:, h * HD:(h + 1) * HD] = sa * xv[:, h * HD:(h + 1) * HD] + sb * vm


def _s4(x, wv, bv2, v1, i1, i2, a1, a2, sab):
    return pl.pallas_call(
        _s4_body,
        grid=(NTILE,),
        in_specs=[
            pl.BlockSpec((TILE, DIM), lambda i: (i, 0)),
            pl.BlockSpec((DIM, DIM), lambda i: (0, 0)),
            pl.BlockSpec((1, DIM), lambda i: (0, 0)),
            pl.BlockSpec((HP, HD), lambda i: (0, 0)),
            pl.BlockSpec((TILE, H), lambda i: (i, 0)),
            pl.BlockSpec((TILE, H), lambda i: (i, 0)),
            pl.BlockSpec((TILE, H), lambda i: (i, 0)),
            pl.BlockSpec((TILE, H), lambda i: (i, 0)),
            pl.BlockSpec((2, H), lambda i: (0, 0)),
        ],
        out_specs=pl.BlockSpec((TILE, DIM), lambda i: (i, 0)),
        out_shape=jax.ShapeDtypeStruct((N, DIM), jnp.float32),
    )(x, wv, bv2, v1, i1, i2, a1, a2, sab)


def kernel(x, bias, Wq, bq, proto_key, Wv, bv, emb, alpha, beta):
    pkt_b = proto_key.transpose(2, 1, 0).reshape(HD, HP).astype(jnp.bfloat16)
    wq_b = Wq.astype(jnp.bfloat16)
    emb_c = emb.reshape(HP)
    sab = jnp.concatenate([jax.nn.sigmoid(alpha).reshape(1, H),
                           jax.nn.sigmoid(beta).reshape(1, H)], axis=0)

    attn_t, segmax_3d, i1, i2, a1, a2 = _s1(x, wq_b, bq.reshape(1, DIM), pkt_b, bias)
    segmax_t = segmax_3d.transpose(1, 0, 2).reshape(HP, NSEG)
    u, ssum = _s2_jax(attn_t, segmax_t, emb_c, x)
    bvtile = jnp.broadcast_to(bv.reshape(H, 1, HD), (H, P, HD)).reshape(HP, HD)
    v1 = _s3(u, Wv, bvtile, ssum.reshape(HP, 1))
    v = _s4(x, Wv.astype(jnp.bfloat16), bv.reshape(1, DIM), v1, i1, i2, a1, a2, sab)
    topk_idx = jnp.stack([i1, i2], axis=-1)
    return v, topk_idx


# speedup vs baseline: 7.8084x; 1.0120x over previous
import functools
import jax
import jax.numpy as jnp
from jax import lax
from jax.experimental import pallas as pl
from jax.experimental.pallas import tpu as pltpu
from jax.experimental.pallas import tpu_sc as plsc

N = 16384
DIM = 768
H = 12
HD = 64
P = 64
HP = H * P
K2 = 2
K1 = 8
SEG = 128
NSEG = N // SEG
TILE = 256
NTILE = N // TILE
SEG_PER_TILE = TILE // SEG

_NEG = -1e30


def _s1_body(x_ref, wq_ref, bq_ref, pkt_ref, bias_ref,
             attn_t_ref, segmax_t_ref, i1_ref, i2_ref, a1_ref, a2_ref):
    x = x_ref[...]
    q = jnp.dot(x.astype(jnp.bfloat16), wq_ref[...],
                preferred_element_type=jnp.float32) + bq_ref[...]
    qb = q.astype(jnp.bfloat16)
    ahs = []
    for h in range(H):
        ah = jnp.dot(qb[:, h * HD:(h + 1) * HD], pkt_ref[:, h * P:(h + 1) * P],
                     preferred_element_type=jnp.float32) * (HD ** -0.5)
        ahs.append(ah)
    a = jnp.concatenate(ahs, axis=1)
    attn_t_ref[...] = a.T
    segmax_t_ref[...] = a.reshape(SEG_PER_TILE, SEG, HP).max(axis=1).T[None]

    iota = lax.broadcasted_iota(jnp.int32, (TILE, P), 1)
    i1s, i2s, a1s, a2s = [], [], [], []
    for h in range(H):
        ah = ahs[h]
        sch = jax.nn.sigmoid(ah) + bias_ref[h:h + 1, :]
        m1 = sch.max(axis=1, keepdims=True)
        i1 = jnp.where(sch == m1, iota, P).min(axis=1, keepdims=True)
        a1 = jnp.where(iota == i1, ah, 0.0).sum(axis=1, keepdims=True)
        sc2 = jnp.where(iota == i1, _NEG, sch)
        m2 = sc2.max(axis=1, keepdims=True)
        i2 = jnp.where(sc2 == m2, iota, P).min(axis=1, keepdims=True)
        a2 = jnp.where(iota == i2, ah, 0.0).sum(axis=1, keepdims=True)
        i1s.append(i1); i2s.append(i2); a1s.append(a1); a2s.append(a2)
    i1_ref[...] = jnp.concatenate(i1s, axis=1)
    i2_ref[...] = jnp.concatenate(i2s, axis=1)
    a1_ref[...] = jnp.concatenate(a1s, axis=1)
    a2_ref[...] = jnp.concatenate(a2s, axis=1)


def _s1(x, wq_b, bq2, pkt_b, bias):
    return pl.pallas_call(
        _s1_body,
        grid=(NTILE,),
        in_specs=[
            pl.BlockSpec((TILE, DIM), lambda i: (i, 0)),
            pl.BlockSpec((DIM, DIM), lambda i: (0, 0)),
            pl.BlockSpec((1, DIM), lambda i: (0, 0)),
            pl.BlockSpec((HD, HP), lambda i: (0, 0)),
            pl.BlockSpec((H, P), lambda i: (0, 0)),
        ],
        out_specs=[
            pl.BlockSpec((HP, TILE), lambda i: (0, i)),
            pl.BlockSpec((1, HP, SEG_PER_TILE), lambda i: (i, 0, 0)),
            pl.BlockSpec((TILE, H), lambda i: (i, 0)),
            pl.BlockSpec((TILE, H), lambda i: (i, 0)),
            pl.BlockSpec((TILE, H), lambda i: (i, 0)),
            pl.BlockSpec((TILE, H), lambda i: (i, 0)),
        ],
        out_shape=[
            jax.ShapeDtypeStruct((HP, N), jnp.float32),
            jax.ShapeDtypeStruct((NTILE, HP, SEG_PER_TILE), jnp.float32),
            jax.ShapeDtypeStruct((N, H), jnp.int32),
            jax.ShapeDtypeStruct((N, H), jnp.int32),
            jax.ShapeDtypeStruct((N, H), jnp.float32),
            jax.ShapeDtypeStruct((N, H), jnp.float32),
        ],
    )(x, wq_b, bq2, pkt_b, bias)


_NEGF = -3.0e38


def _vtake(v, idx):
    return v.at[idx].get(mode="promise_in_bounds")


def _bmax(v, lane):
    for s in (8, 4, 2, 1):
        v = jnp.maximum(v, _vtake(v, lane ^ s))
    return v


def _bmin(v, lane):
    for s in (8, 4, 2, 1):
        v = jnp.minimum(v, _vtake(v, lane ^ s))
    return v


def _s2_sc(segmax_t, attn_r, x, emb_c):
    info = plsc.get_sparse_core_info()
    nc, ns = info.num_cores, info.num_subcores
    nw = nc * ns
    cpw = HP // nw
    mesh = plsc.VectorSubcoreMesh(core_axis_name="c", subcore_axis_name="s")

    @functools.partial(
        pl.kernel, mesh=mesh,
        out_type=[jax.ShapeDtypeStruct((HP, DIM), jnp.float32),
                  jax.ShapeDtypeStruct((HP, 16), jnp.float32)],
        scratch_types=[
            pltpu.VMEM((NSEG,), jnp.float32),
            pltpu.VMEM((16,), jnp.int32),
            pltpu.VMEM((16, SEG), jnp.float32),
            pltpu.VMEM((16,), jnp.int32),
            pltpu.VMEM((16, DIM), jnp.float32),
            pltpu.VMEM((DIM,), jnp.float32),
            pltpu.VMEM((16,), jnp.float32),
            pltpu.VMEM((HP,), jnp.float32),
            pltpu.SemaphoreType.DMA,
        ],
        compiler_params=pltpu.CompilerParams(needs_layout_passes=False))
    def s2(segmax_hbm, attn_hbm, x_hbm, emb_hbm, u_hbm, w_hbm,
           seg_v, cidx_v, chunk_v, tidx_v, xrows_v, urow_v, wv_v, emb_v,
           sem):
        wid = lax.axis_index("s") * nc + lax.axis_index("c")
        base = wid * cpw
        lane = lax.iota(jnp.int32, 16)
        pltpu.sync_copy(emb_hbm, emb_v)

        def col_body(k, _):
            c = base + k
            pltpu.sync_copy(segmax_hbm.at[c], seg_v)

            cidx_acc = jnp.zeros((16,), jnp.int32)
            sseg_acc = jnp.zeros((16,), jnp.int32)
            negv = jnp.full((16,), _NEGF, jnp.float32)
            for i in range(K1):
                mval = negv
                midx = jnp.zeros((16,), jnp.int32)
                for j in range(NSEG // 16):
                    v = seg_v[pl.ds(16 * j, 16)]
                    take = v > mval
                    mval = jnp.where(take, v, mval)
                    midx = jnp.where(take, lane + 16 * j, midx)
                mbest = _bmax(mval, lane)
                sidx = _bmin(jnp.where(mval == mbest, midx, NSEG), lane)
                plsc.store_scatter(seg_v, [sidx], negv, mask=lane == 0)
                sseg_acc = jnp.where(lane == i, sidx, sseg_acc)
                cidx_acc = jnp.where(lane == i, c * NSEG + sidx, cidx_acc)
            cidx_v[...] = cidx_acc
            pltpu.async_copy(attn_hbm.at[cidx_v], chunk_v, sem).wait()

            tidx_acc = jnp.zeros((16,), jnp.int32)
            tval_acc = jnp.zeros((16,), jnp.float32)
            for i in range(K1):
                mval = negv
                mpos = jnp.zeros((16,), jnp.int32)
                for rj in range(K1):
                    for kk in range(SEG // 16):
                        v = chunk_v[rj, pl.ds(16 * kk, 16)]
                        take = v > mval
                        mval = jnp.where(take, v, mval)
                        mpos = jnp.where(take, rj * SEG + 16 * kk + lane, mpos)
                best = _bmax(mval, lane)
                fpos = _bmin(jnp.where(mval == best, mpos, K1 * SEG), lane)
                plsc.store_scatter(chunk_v, [fpos // SEG, fpos % SEG],
                                   negv, mask=lane == 0)
                g = _vtake(sseg_acc, fpos // SEG) * SEG + fpos % SEG
                tidx_acc = jnp.where(lane == i, g, tidx_acc)
                tval_acc = jnp.where(lane == i, best, tval_acc)
            tidx_v[...] = tidx_acc
            pltpu.async_copy(x_hbm.at[tidx_v], xrows_v, sem).wait()

            ev = emb_v[pl.ds((c // 16) * 16, 16)]
            esv = _bmax(jnp.where(lane == c % 16, ev, _NEGF), lane)
            wvec = 1.0 / (1.0 + jnp.exp(-(tval_acc + esv)))
            wvec = jnp.where(lane < K1, wvec, 0.0)
            wv_v[...] = wvec
            ws = [_vtake(wvec, jnp.full((16,), i, jnp.int32))
                  for i in range(K1)]

            def acc_body(j, _):
                acc = ws[0] * xrows_v[0, pl.ds(16 * j, 16)]
                for i in range(1, K1):
                    acc = acc + ws[i] * xrows_v[i, pl.ds(16 * j, 16)]
                urow_v[pl.ds(16 * j, 16)] = acc
                return _

            lax.fori_loop(0, DIM // 16, acc_body, None)
            pltpu.sync_copy(urow_v, u_hbm.at[c])
            pltpu.sync_copy(wv_v, w_hbm.at[c])
            return _

        lax.fori_loop(0, cpw, col_body, None)

    return s2(segmax_t, attn_r, x, emb_c)


def _s3_body(u_ref, wv_ref, bvtile_ref, wmat_ref, v1_ref):
    full = jnp.dot(u_ref[...], wv_ref[...], preferred_element_type=jnp.float32)
    blk = jnp.concatenate(
        [full[h * P:(h + 1) * P, h * HD:(h + 1) * HD] for h in range(H)], axis=0)
    ssum = jnp.sum(wmat_ref[...], axis=1, keepdims=True)
    v1_ref[...] = blk + ssum * bvtile_ref[...]


def _s3(u, wv, bvtile, wmat):
    return pl.pallas_call(
        _s3_body,
        in_specs=[
            pl.BlockSpec((HP, DIM), lambda: (0, 0)),
            pl.BlockSpec((DIM, DIM), lambda: (0, 0)),
            pl.BlockSpec((HP, HD), lambda: (0, 0)),
            pl.BlockSpec((HP, 16), lambda: (0, 0)),
        ],
        out_specs=pl.BlockSpec((HP, HD), lambda: (0, 0)),
        out_shape=jax.ShapeDtypeStruct((HP, HD), jnp.float32),
    )(u, wv, bvtile, wmat)


def _s4_body(x_ref, wv_ref, bv2_ref, v1_ref, i1_ref, i2_ref, a1_ref, a2_ref,
             sab_ref, out_ref):
    x = x_ref[...]
    xv = jnp.dot(x.astype(jnp.bfloat16), wv_ref[...],
                 preferred_element_type=jnp.float32) + bv2_ref[...]
    iota = lax.broadcasted_iota(jnp.int32, (TILE, P), 1)
    for h in range(H):
        i1 = i1_ref[:, h:h + 1]
        i2 = i2_ref[:, h:h + 1]
        a1 = a1_ref[:, h:h + 1]
        a2 = a2_ref[:, h:h + 1]
        mx = jnp.maximum(a1, a2)
        e1 = jnp.exp(a1 - mx)
        e2 = jnp.exp(a2 - mx)
        inv = 1.0 / (e1 + e2)
        w1 = e1 * inv
        w2 = e2 * inv
        whot = jnp.where(iota == i1, w1, 0.0) + jnp.where(iota == i2, w2, 0.0)
        vm = jnp.dot(whot.astype(jnp.bfloat16),
                     v1_ref[h * P:(h + 1) * P, :].astype(jnp.bfloat16),
                     preferred_element_type=jnp.float32)
        sa = sab_ref[0:1, h:h + 1]
        sb = sab_ref[1:2, h:h + 1]
        out_ref[:, h * HD:(h + 1) * HD] = sa * xv[:, h * HD:(h + 1) * HD] + sb * vm


def _s4(x, wv, bv2, v1, i1, i2, a1, a2, sab):
    return pl.pallas_call(
        _s4_body,
        grid=(NTILE,),
        in_specs=[
            pl.BlockSpec((TILE, DIM), lambda i: (i, 0)),
            pl.BlockSpec((DIM, DIM), lambda i: (0, 0)),
            pl.BlockSpec((1, DIM), lambda i: (0, 0)),
            pl.BlockSpec((HP, HD), lambda i: (0, 0)),
            pl.BlockSpec((TILE, H), lambda i: (i, 0)),
            pl.BlockSpec((TILE, H), lambda i: (i, 0)),
            pl.BlockSpec((TILE, H), lambda i: (i, 0)),
            pl.BlockSpec((TILE, H), lambda i: (i, 0)),
            pl.BlockSpec((2, H), lambda i: (0, 0)),
        ],
        out_specs=pl.BlockSpec((TILE, DIM), lambda i: (i, 0)),
        out_shape=jax.ShapeDtypeStruct((N, DIM), jnp.float32),
    )(x, wv, bv2, v1, i1, i2, a1, a2, sab)


def kernel(x, bias, Wq, bq, proto_key, Wv, bv, emb, alpha, beta):
    pkt_b = proto_key.transpose(2, 1, 0).reshape(HD, HP).astype(jnp.bfloat16)
    wq_b = Wq.astype(jnp.bfloat16)
    emb_c = emb.reshape(HP)
    sab = jnp.concatenate([jax.nn.sigmoid(alpha).reshape(1, H),
                           jax.nn.sigmoid(beta).reshape(1, H)], axis=0)

    attn_t, segmax_3d, i1, i2, a1, a2 = _s1(x, wq_b, bq.reshape(1, DIM), pkt_b, bias)
    segmax_t = segmax_3d.transpose(1, 0, 2).reshape(HP, NSEG)
    attn_r = attn_t.reshape(HP * NSEG, SEG)
    u, wmat = _s2_sc(segmax_t, attn_r, x, emb_c)
    bvtile = jnp.broadcast_to(bv.reshape(H, 1, HD), (H, P, HD)).reshape(HP, HD)
    v1 = _s3(u, Wv, bvtile, wmat)
    v = _s4(x, Wv.astype(jnp.bfloat16), bv.reshape(1, DIM), v1, i1, i2, a1, a2, sab)
    topk_idx = jnp.stack([i1, i2], axis=-1)
    return v, topk_idx


# speedup vs baseline: 11.5404x; 1.4780x over previous
import functools
import jax
import jax.numpy as jnp
from jax import lax
from jax.experimental import pallas as pl
from jax.experimental.pallas import tpu as pltpu
from jax.experimental.pallas import tpu_sc as plsc

N = 16384
DIM = 768
H = 12
HD = 64
P = 64
HP = H * P
K2 = 2
K1 = 8
SEG = 128
NSEG = N // SEG
TILE = 256
NTILE = N // TILE
SEG_PER_TILE = TILE // SEG

_NEG = -1e30


def _s1_body(x_ref, wq_ref, bq_ref, pkt_ref, bias_ref,
             attn_t_ref, segmax_t_ref, i1_ref, i2_ref, a1_ref, a2_ref):
    x = x_ref[...]
    q = jnp.dot(x.astype(jnp.bfloat16), wq_ref[...],
                preferred_element_type=jnp.float32) + bq_ref[...]
    qb = q.astype(jnp.bfloat16)
    ahs = []
    for h in range(H):
        ah = jnp.dot(qb[:, h * HD:(h + 1) * HD], pkt_ref[:, h * P:(h + 1) * P],
                     preferred_element_type=jnp.float32) * (HD ** -0.5)
        ahs.append(ah)
    a = jnp.concatenate(ahs, axis=1)
    attn_t_ref[...] = a.T
    segmax_t_ref[...] = a.reshape(SEG_PER_TILE, SEG, HP).max(axis=1).T[None]

    iota = lax.broadcasted_iota(jnp.int32, (TILE, P), 1)
    i1s, i2s, a1s, a2s = [], [], [], []
    for h in range(H):
        ah = ahs[h]
        sch = jax.nn.sigmoid(ah) + bias_ref[h:h + 1, :]
        m1 = sch.max(axis=1, keepdims=True)
        i1 = jnp.where(sch == m1, iota, P).min(axis=1, keepdims=True)
        a1 = jnp.where(iota == i1, ah, 0.0).sum(axis=1, keepdims=True)
        sc2 = jnp.where(iota == i1, _NEG, sch)
        m2 = sc2.max(axis=1, keepdims=True)
        i2 = jnp.where(sc2 == m2, iota, P).min(axis=1, keepdims=True)
        a2 = jnp.where(iota == i2, ah, 0.0).sum(axis=1, keepdims=True)
        i1s.append(i1); i2s.append(i2); a1s.append(a1); a2s.append(a2)
    i1_ref[...] = jnp.concatenate(i1s, axis=1)
    i2_ref[...] = jnp.concatenate(i2s, axis=1)
    a1_ref[...] = jnp.concatenate(a1s, axis=1)
    a2_ref[...] = jnp.concatenate(a2s, axis=1)


def _s1(x, wq_b, bq2, pkt_b, bias):
    return pl.pallas_call(
        _s1_body,
        grid=(NTILE,),
        in_specs=[
            pl.BlockSpec((TILE, DIM), lambda i: (i, 0)),
            pl.BlockSpec((DIM, DIM), lambda i: (0, 0)),
            pl.BlockSpec((1, DIM), lambda i: (0, 0)),
            pl.BlockSpec((HD, HP), lambda i: (0, 0)),
            pl.BlockSpec((H, P), lambda i: (0, 0)),
        ],
        out_specs=[
            pl.BlockSpec((HP, TILE), lambda i: (0, i)),
            pl.BlockSpec((1, HP, SEG_PER_TILE), lambda i: (i, 0, 0)),
            pl.BlockSpec((TILE, H), lambda i: (i, 0)),
            pl.BlockSpec((TILE, H), lambda i: (i, 0)),
            pl.BlockSpec((TILE, H), lambda i: (i, 0)),
            pl.BlockSpec((TILE, H), lambda i: (i, 0)),
        ],
        out_shape=[
            jax.ShapeDtypeStruct((HP, N), jnp.float32),
            jax.ShapeDtypeStruct((NTILE, HP, SEG_PER_TILE), jnp.float32),
            jax.ShapeDtypeStruct((N, H), jnp.int32),
            jax.ShapeDtypeStruct((N, H), jnp.int32),
            jax.ShapeDtypeStruct((N, H), jnp.float32),
            jax.ShapeDtypeStruct((N, H), jnp.float32),
        ],
    )(x, wq_b, bq2, pkt_b, bias)


_NEGF = -3.0e38


def _vtake(v, idx):
    return v.at[idx].get(mode="promise_in_bounds")


def _bmax(v, lane):
    for s in (8, 4, 2, 1):
        v = jnp.maximum(v, _vtake(v, lane ^ s))
    return v


def _bmin(v, lane):
    for s in (8, 4, 2, 1):
        v = jnp.minimum(v, _vtake(v, lane ^ s))
    return v


def _s2_sc(segmax_t, attn_r, x, emb_c):
    info = plsc.get_sparse_core_info()
    nc, ns = info.num_cores, info.num_subcores
    nw = nc * ns
    cpw = HP // nw
    mesh = plsc.VectorSubcoreMesh(core_axis_name="c", subcore_axis_name="s")

    @functools.partial(
        pl.kernel, mesh=mesh,
        out_type=[jax.ShapeDtypeStruct((HP, DIM), jnp.float32),
                  jax.ShapeDtypeStruct((HP, 16), jnp.float32)],
        scratch_types=[
            pltpu.VMEM((cpw, NSEG), jnp.float32),
            pltpu.VMEM((cpw * 16,), jnp.int32),
            pltpu.VMEM((cpw * K1, SEG), jnp.float32),
            pltpu.VMEM((cpw * 16,), jnp.int32),
            pltpu.VMEM((2 * K1, DIM), jnp.float32),
            pltpu.VMEM((cpw, DIM), jnp.float32),
            pltpu.VMEM((cpw, 16), jnp.float32),
            pltpu.VMEM((HP,), jnp.float32),
            pltpu.SemaphoreType.DMA,
            pltpu.SemaphoreType.DMA,
        ],
        compiler_params=pltpu.CompilerParams(needs_layout_passes=False))
    def s2(segmax_hbm, attn_hbm, x_hbm, emb_hbm, u_hbm, w_hbm,
           seg_all, c_all, chunks, t_all, xr, u_all, w_all, emb_v,
           sem_c, sem_x):
        wid = lax.axis_index("s") * nc + lax.axis_index("c")
        base = wid * cpw
        lane = lax.iota(jnp.int32, 16)
        negv = jnp.full((16,), _NEGF, jnp.float32)
        pltpu.sync_copy(emb_hbm, emb_v)
        pltpu.sync_copy(segmax_hbm.at[pl.ds(base, cpw)], seg_all)

        def p1(k, _):
            c = base + k
            cidx_acc = jnp.zeros((16,), jnp.int32)

            def seg_scan(j, carry):
                mval, midx = carry
                for s in range(4):
                    v = seg_all[k, pl.ds(64 * j + 16 * s, 16)]
                    take = v > mval
                    mval = jnp.where(take, v, mval)
                    midx = jnp.where(take, lane + 64 * j + 16 * s, midx)
                return mval, midx

            for i in range(K1):
                mval, midx = lax.fori_loop(
                    0, NSEG // 64, seg_scan,
                    (negv, jnp.zeros((16,), jnp.int32)))
                mbest = _bmax(mval, lane)
                sidx = _bmin(jnp.where(mval == mbest, midx, NSEG), lane)
                plsc.store_scatter(seg_all,
                                   [jnp.broadcast_to(k, (16,)), sidx],
                                   negv, mask=lane == 0)
                cidx_acc = jnp.where(lane == i, c * NSEG + sidx, cidx_acc)
            c_all[pl.ds(16 * k, 16)] = cidx_acc
            pltpu.async_copy(attn_hbm.at[c_all.at[pl.ds(16 * k, K1)]],
                             chunks.at[pl.ds(K1 * k, K1)], sem_c)
            return _

        lax.fori_loop(0, cpw, p1, None)

        def accum(kk):
            slot = lax.rem(kk, 2)
            pltpu.make_async_copy(x_hbm.at[pl.ds(0, K1)],
                                  xr.at[pl.ds(K1 * slot, K1)], sem_x).wait()
            wv = w_all[kk, pl.ds(0, 16)]
            ws = [_vtake(wv, jnp.full((16,), i, jnp.int32)) for i in range(K1)]

            def acc_body(j, _):
                acc = ws[0] * xr[K1 * slot, pl.ds(16 * j, 16)]
                for i in range(1, K1):
                    acc = acc + ws[i] * xr[K1 * slot + i, pl.ds(16 * j, 16)]
                u_all[kk, pl.ds(16 * j, 16)] = acc
                return _

            lax.fori_loop(0, DIM // 16, acc_body, None)

        def p2(k, _):
            c = base + k
            pltpu.make_async_copy(attn_hbm.at[pl.ds(0, K1)],
                                  chunks.at[pl.ds(K1 * k, K1)], sem_c).wait()
            sseg = c_all[pl.ds(16 * k, 16)] - c * NSEG
            tidx_acc = jnp.zeros((16,), jnp.int32)
            tval_acc = jnp.zeros((16,), jnp.float32)

            def chunk_scan(rj, carry):
                mval, mpos = carry
                for kk in range(SEG // 16):
                    v = chunks[K1 * k + rj, pl.ds(16 * kk, 16)]
                    take = v > mval
                    mval = jnp.where(take, v, mval)
                    mpos = jnp.where(take, rj * SEG + 16 * kk + lane, mpos)
                return mval, mpos

            for i in range(K1):
                mval, mpos = lax.fori_loop(
                    0, K1, chunk_scan, (negv, jnp.zeros((16,), jnp.int32)))
                best = _bmax(mval, lane)
                fpos = _bmin(jnp.where(mval == best, mpos, K1 * SEG), lane)
                plsc.store_scatter(chunks,
                                   [K1 * k + fpos // SEG, fpos % SEG],
                                   negv, mask=lane == 0)
                g = _vtake(sseg, fpos // SEG) * SEG + fpos % SEG
                tidx_acc = jnp.where(lane == i, g, tidx_acc)
                tval_acc = jnp.where(lane == i, best, tval_acc)
            t_all[pl.ds(16 * k, 16)] = tidx_acc

            ev = emb_v[pl.ds((c // 16) * 16, 16)]
            esv = _bmax(jnp.where(lane == c % 16, ev, _NEGF), lane)
            wvec = 1.0 / (1.0 + jnp.exp(-(tval_acc + esv)))
            w_all[k, pl.ds(0, 16)] = jnp.where(lane < K1, wvec, 0.0)

            slot = lax.rem(k, 2)
            pltpu.async_copy(x_hbm.at[t_all.at[pl.ds(16 * k, K1)]],
                             xr.at[pl.ds(K1 * slot, K1)], sem_x)

            @pl.when(k >= 1)
            def _older():
                accum(k - 1)
            return _

        lax.fori_loop(0, cpw, p2, None)
        accum(cpw - 1)
        pltpu.sync_copy(u_all, u_hbm.at[pl.ds(base, cpw)])
        pltpu.sync_copy(w_all, w_hbm.at[pl.ds(base, cpw)])

    return s2(segmax_t, attn_r, x, emb_c)


def _s3_body(u_ref, wv_ref, bvtile_ref, wmat_ref, v1_ref):
    full = jnp.dot(u_ref[...], wv_ref[...], preferred_element_type=jnp.float32)
    blk = jnp.concatenate(
        [full[h * P:(h + 1) * P, h * HD:(h + 1) * HD] for h in range(H)], axis=0)
    ssum = jnp.sum(wmat_ref[...], axis=1, keepdims=True)
    v1_ref[...] = blk + ssum * bvtile_ref[...]


def _s3(u, wv, bvtile, wmat):
    return pl.pallas_call(
        _s3_body,
        in_specs=[
            pl.BlockSpec((HP, DIM), lambda: (0, 0)),
            pl.BlockSpec((DIM, DIM), lambda: (0, 0)),
            pl.BlockSpec((HP, HD), lambda: (0, 0)),
            pl.BlockSpec((HP, 16), lambda: (0, 0)),
        ],
        out_specs=pl.BlockSpec((HP, HD), lambda: (0, 0)),
        out_shape=jax.ShapeDtypeStruct((HP, HD), jnp.float32),
    )(u, wv, bvtile, wmat)


def _s4_body(x_ref, wv_ref, bv2_ref, v1_ref, i1_ref, i2_ref, a1_ref, a2_ref,
             sab_ref, out_ref):
    x = x_ref[...]
    xv = jnp.dot(x.astype(jnp.bfloat16), wv_ref[...],
                 preferred_element_type=jnp.float32) + bv2_ref[...]
    iota = lax.broadcasted_iota(jnp.int32, (TILE, P), 1)
    for h in range(H):
        i1 = i1_ref[:, h:h + 1]
        i2 = i2_ref[:, h:h + 1]
        a1 = a1_ref[:, h:h + 1]
        a2 = a2_ref[:, h:h + 1]
        mx = jnp.maximum(a1, a2)
        e1 = jnp.exp(a1 - mx)
        e2 = jnp.exp(a2 - mx)
        inv = 1.0 / (e1 + e2)
        w1 = e1 * inv
        w2 = e2 * inv
        whot = jnp.where(iota == i1, w1, 0.0) + jnp.where(iota == i2, w2, 0.0)
        vm = jnp.dot(whot.astype(jnp.bfloat16),
                     v1_ref[h * P:(h + 1) * P, :].astype(jnp.bfloat16),
                     preferred_element_type=jnp.float32)
        sa = sab_ref[0:1, h:h + 1]
        sb = sab_ref[1:2, h:h + 1]
        out_ref[:, h * HD:(h + 1) * HD] = sa * xv[:, h * HD:(h + 1) * HD] + sb * vm


def _s4(x, wv, bv2, v1, i1, i2, a1, a2, sab):
    return pl.pallas_call(
        _s4_body,
        grid=(NTILE,),
        in_specs=[
            pl.BlockSpec((TILE, DIM), lambda i: (i, 0)),
            pl.BlockSpec((DIM, DIM), lambda i: (0, 0)),
            pl.BlockSpec((1, DIM), lambda i: (0, 0)),
            pl.BlockSpec((HP, HD), lambda i: (0, 0)),
            pl.BlockSpec((TILE, H), lambda i: (i, 0)),
            pl.BlockSpec((TILE, H), lambda i: (i, 0)),
            pl.BlockSpec((TILE, H), lambda i: (i, 0)),
            pl.BlockSpec((TILE, H), lambda i: (i, 0)),
            pl.BlockSpec((2, H), lambda i: (0, 0)),
        ],
        out_specs=pl.BlockSpec((TILE, DIM), lambda i: (i, 0)),
        out_shape=jax.ShapeDtypeStruct((N, DIM), jnp.float32),
    )(x, wv, bv2, v1, i1, i2, a1, a2, sab)


def kernel(x, bias, Wq, bq, proto_key, Wv, bv, emb, alpha, beta):
    pkt_b = proto_key.transpose(2, 1, 0).reshape(HD, HP).astype(jnp.bfloat16)
    wq_b = Wq.astype(jnp.bfloat16)
    emb_c = emb.reshape(HP)
    sab = jnp.concatenate([jax.nn.sigmoid(alpha).reshape(1, H),
                           jax.nn.sigmoid(beta).reshape(1, H)], axis=0)

    attn_t, segmax_3d, i1, i2, a1, a2 = _s1(x, wq_b, bq.reshape(1, DIM), pkt_b, bias)
    segmax_t = segmax_3d.transpose(1, 0, 2).reshape(HP, NSEG)
    attn_r = attn_t.reshape(HP * NSEG, SEG)
    u, wmat = _s2_sc(segmax_t, attn_r, x, emb_c)
    bvtile = jnp.broadcast_to(bv.reshape(H, 1, HD), (H, P, HD)).reshape(HP, HD)
    v1 = _s3(u, Wv, bvtile, wmat)
    v = _s4(x, Wv.astype(jnp.bfloat16), bv.reshape(1, DIM), v1, i1, i2, a1, a2, sab)
    topk_idx = jnp.stack([i1, i2], axis=-1)
    return v, topk_idx
